# baseline (speedup 1.0000x reference)
import functools

import jax
import jax.numpy as jnp
from jax import lax
from jax.experimental import pallas as pl
from jax.experimental.pallas import tpu as pltpu
from jax.experimental.pallas import tpu_sc as plsc

N_USERS_P1 = 1000001
N_CLUSTERS = 64
USER_EMBED_DIM = 64
BATCH = 16384

NC = 2
NS = 16
NW = NC * NS

SB = 512
HB = 256
N_FULL_SB = 999424 // SB
SB_PER_W = N_FULL_SB // NW
W_COLS = SB_PER_W * SB
TAIL_START = N_FULL_SB * SB
TAIL_N = N_USERS_P1 - TAIL_START
TAIL_Q = 2 * SB_PER_W

POS_BITS = 14
CAP = BATCH + 16
SEG = 2048
SCAP = SEG + 16
NSLOT = 32


def _sc_body(idx_hbm, table_hbm, tail_hbm, out_hbm,
             hits_v, sbh_v, bufa_v, bufb_v, tailb_v, slots_v,
             sema, semb, sem_out):
  wid = lax.axis_index("s") * NC + lax.axis_index("c")
  lo_col = wid * W_COLS
  is_last = wid == NW - 1
  hi_col = jnp.where(is_last, N_USERS_P1, lo_col + W_COLS)

  lane = lax.iota(jnp.int32, 16)
  cvecs = [lane + 16 * k for k in range(4)]

  pltpu.sync_copy(idx_hbm, hits_v.at[pl.ds(0, BATCH)])

  def scan_body(g, cursor):
    idv = hits_v[pl.ds(16 * g, 16)]
    m = (idv >= lo_col) & (idv < hi_col)
    rel = idv - lo_col
    packed = rel * (1 << POS_BITS) + (16 * g + lane)
    plsc.store_compressed(hits_v.at[pl.ds(cursor, 16)], packed, mask=m)
    cnt = plsc.all_reduce_population_count(m)[0]
    return cursor + cnt

  nh = lax.fori_loop(0, BATCH // 16, scan_body, 0)
  nseg = (nh + SEG - 1) // SEG

  def match_q(q, seg0, segn, ge=False):
    def mbody(g, c2):
      hv = hits_v[pl.ds(seg0 + 16 * g, 16)]
      valid = (16 * g + lane) < segn
      qq = hv >> (POS_BITS + 8)
      mm = ((qq >= q) if ge else (qq == q)) & valid
      plsc.store_compressed(sbh_v.at[pl.ds(c2, 16)], hv, mask=mm)
      return c2 + plsc.all_reduce_population_count(mm)[0]

    return lax.fori_loop(0, (segn + 15) // 16, mbody, 0)

  def extract_hits(c2, src_ref, is_tail, k0):
    def ebody(g, k):
      hv = sbh_v[pl.ds(16 * g, 16)]
      for r in range(16):
        @pl.when(16 * g + r < c2)
        def _():
          w = hv[r]
          rel = w >> POS_BITS
          pos = w & ((1 << POS_BITS) - 1)
          if is_tail:
            m = rel - (TAIL_Q << 8)
          else:
            m = rel & (HB - 1)
          kk = k + r
          slot = kk & (NSLOT - 1)

          @pl.when((kk >= NSLOT) & (slot == 0))
          def _():
            pltpu.make_async_copy(
                out_hbm.at[pl.ds(0, NSLOT), :], slots_v, sem_out).wait()

          for k4 in range(4):
            if is_tail:
              v = plsc.load_gather(src_ref, [jnp.full((16,), m, jnp.int32),
                                             cvecs[k4]])
            else:
              v = plsc.load_gather(src_ref, [cvecs[k4],
                                             jnp.full((16,), m, jnp.int32)])
            slots_v[slot, pl.ds(16 * k4, 16)] = v
          pltpu.async_copy(slots_v.at[pl.ds(slot, 1), :],
                           out_hbm.at[pl.ds(pos, 1), :], sem_out)

      return k + 16

    lax.fori_loop(0, (c2 + 15) // 16, ebody, k0)
    def dbody(i, _):
      pltpu.make_async_copy(
          out_hbm.at[pl.ds(0, 1), :], slots_v.at[pl.ds(0, 1), :],
          sem_out).wait()
      return 0

    drained = ((c2 - 1) // NSLOT) * NSLOT
    lax.fori_loop(0, c2 - drained, dbody, 0)

  def match_extract(q, src_ref, is_tail):
    def seg_body(s, _):
      seg0 = s * SEG
      segn = jnp.minimum(nh - seg0, SEG)
      c2 = match_q(q, seg0, segn, ge=is_tail)

      @pl.when(c2 > 0)
      def _():
        extract_hits(c2, src_ref, is_tail, 0)

      return 0

    lax.fori_loop(0, nseg, seg_body, 0)

  bufs = (bufa_v, bufb_v)
  sems = (sema, semb)

  def fire_half(hsb, half):
    start = pl.multiple_of(lo_col + hsb * HB, HB)
    for q4 in range(2):
      pltpu.async_copy(
          table_hbm.at[:, pl.ds(start + q4 * (HB // 2), HB // 2)],
          bufs[half].at[:, pl.ds(q4 * (HB // 2), HB // 2)], sems[half])

  fire_half(0, 0)
  fire_half(1, 1)

  def sweep_body(sb, _):
    for half in range(2):
      pltpu.make_async_copy(table_hbm.at[:, pl.ds(0, HB)],
                            bufs[half], sems[half]).wait()
      match_extract(2 * sb + half, bufs[half], False)

      @pl.when(sb < SB_PER_W - 1)
      def _():
        fire_half(2 * (sb + 1) + half, half)

    return 0

  lax.fori_loop(0, SB_PER_W, sweep_body, 0)

  @pl.when(is_last)
  def _():
    pltpu.sync_copy(tail_hbm, tailb_v)
    match_extract(TAIL_Q, tailb_v, True)


_sc_gather = pl.kernel(
    _sc_body,
    out_type=jax.ShapeDtypeStruct((BATCH, USER_EMBED_DIM), jnp.float32),
    mesh=plsc.VectorSubcoreMesh(core_axis_name="c", subcore_axis_name="s"),
    compiler_params=pltpu.CompilerParams(needs_layout_passes=False),
    scratch_types=[
        pltpu.VMEM((CAP,), jnp.int32),
        pltpu.VMEM((SCAP,), jnp.int32),
        pltpu.VMEM((N_CLUSTERS, HB), jnp.float32),
        pltpu.VMEM((N_CLUSTERS, HB), jnp.float32),
        pltpu.VMEM((TAIL_N, USER_EMBED_DIM), jnp.float32),
        pltpu.VMEM((NSLOT, USER_EMBED_DIM), jnp.float32),
        pltpu.SemaphoreType.DMA,
        pltpu.SemaphoreType.DMA,
        pltpu.SemaphoreType.DMA,
    ],
)


def _tc_body(rows_ref, ce_ref, out_ref):
  w = rows_ref[...]
  w = w - jnp.max(w, axis=1, keepdims=True)
  e = jnp.exp(w)
  p = e / jnp.sum(e, axis=1, keepdims=True)
  out_ref[...] = jnp.dot(p, ce_ref[...], preferred_element_type=jnp.float32)


TC_BLOCK = 2048


def _tc_softmax_matmul(rows, ce):
  grid = BATCH // TC_BLOCK
  return pl.pallas_call(
      _tc_body,
      grid=(grid,),
      in_specs=[
          pl.BlockSpec((TC_BLOCK, N_CLUSTERS), lambda i: (i, 0)),
          pl.BlockSpec((N_CLUSTERS, USER_EMBED_DIM), lambda i: (0, 0)),
      ],
      out_specs=pl.BlockSpec((TC_BLOCK, USER_EMBED_DIM), lambda i: (i, 0)),
      out_shape=jax.ShapeDtypeStruct((BATCH, USER_EMBED_DIM), jnp.float32),
  )(rows, ce)


@jax.jit
def kernel(user_identifiers, user_weight, cluster_embedding):
  idx = user_identifiers.astype(jnp.int32)
  table_t = jnp.swapaxes(user_weight, 0, 1)
  wt_tail = lax.slice(user_weight, (TAIL_START, 0), (N_USERS_P1, N_CLUSTERS))
  rows = _sc_gather(idx, table_t, wt_tail)
  out = _tc_softmax_matmul(rows, cluster_embedding)
  return out.reshape(BATCH, 1, USER_EMBED_DIM)


# speedup vs baseline: 1.0390x; 1.0390x over previous
import functools

import jax
import jax.numpy as jnp
from jax import lax
from jax.experimental import pallas as pl
from jax.experimental.pallas import tpu as pltpu
from jax.experimental.pallas import tpu_sc as plsc

N_USERS_P1 = 1000001
N_CLUSTERS = 64
USER_EMBED_DIM = 64
BATCH = 16384

NC = 2
NS = 16
NW = NC * NS

SB = 512
HB = 256
N_FULL_SB = 999424 // SB
SB_PER_W = N_FULL_SB // NW
W_COLS = SB_PER_W * SB
TAIL_START = N_FULL_SB * SB
TAIL_N = N_USERS_P1 - TAIL_START
TAIL_Q = 2 * SB_PER_W

POS_BITS = 14
CAP = BATCH + 16
SEG = 2048
SCAP = SEG + 16
NSLOT = 32


def _sc_body(idx_hbm, table_hbm, tail_hbm, out_hbm,
             hits_v, sbh_v, bufa_v, bufb_v, tailb_v, slots_v,
             sema, semb, sem_out):
  wid = lax.axis_index("s") * NC + lax.axis_index("c")
  lo_col = wid * W_COLS
  is_last = wid == NW - 1
  hi_col = jnp.where(is_last, N_USERS_P1, lo_col + W_COLS)

  lane = lax.iota(jnp.int32, 16)
  cvecs = [lane + 16 * k for k in range(4)]

  pltpu.sync_copy(idx_hbm, hits_v.at[pl.ds(0, BATCH)])

  def scan_body(g, cursor):
    idv = hits_v[pl.ds(16 * g, 16)]
    m = (idv >= lo_col) & (idv < hi_col)
    rel = idv - lo_col
    packed = rel * (1 << POS_BITS) + (16 * g + lane)
    plsc.store_compressed(hits_v.at[pl.ds(cursor, 16)], packed, mask=m)
    cnt = plsc.all_reduce_population_count(m)[0]
    return cursor + cnt

  nh = lax.fori_loop(0, BATCH // 16, scan_body, 0)
  nseg = (nh + SEG - 1) // SEG

  def match_q(q, seg0, segn, ge=False):
    def mbody(g, c2):
      hv = hits_v[pl.ds(seg0 + 16 * g, 16)]
      valid = (16 * g + lane) < segn
      qq = hv >> (POS_BITS + 8)
      mm = ((qq >= q) if ge else (qq == q)) & valid
      plsc.store_compressed(sbh_v.at[pl.ds(c2, 16)], hv, mask=mm)
      return c2 + plsc.all_reduce_population_count(mm)[0]

    return lax.fori_loop(0, (segn + 15) // 16, mbody, 0)

  def extract_hits(c2, src_ref, is_tail, k0):
    def ebody(g, k):
      hv = sbh_v[pl.ds(16 * g, 16)]
      for r in range(16):
        @pl.when(16 * g + r < c2)
        def _():
          w = hv[r]
          rel = w >> POS_BITS
          pos = w & ((1 << POS_BITS) - 1)
          if is_tail:
            m = rel - (TAIL_Q << 8)
          else:
            m = rel & (HB - 1)
          kk = k + r
          slot = kk & (NSLOT - 1)

          @pl.when((kk >= NSLOT) & (slot == 0))
          def _():
            pltpu.make_async_copy(
                out_hbm.at[pl.ds(0, NSLOT), :], slots_v, sem_out).wait()

          for k4 in range(4):
            if is_tail:
              v = plsc.load_gather(src_ref, [jnp.full((16,), m, jnp.int32),
                                             cvecs[k4]])
            else:
              v = plsc.load_gather(src_ref, [cvecs[k4],
                                             jnp.full((16,), m, jnp.int32)])
            slots_v[slot, pl.ds(16 * k4, 16)] = v
          pltpu.async_copy(slots_v.at[pl.ds(slot, 1), :],
                           out_hbm.at[pl.ds(pos, 1), :], sem_out)

      return k + 16

    lax.fori_loop(0, (c2 + 15) // 16, ebody, k0)
    def dbody(i, _):
      pltpu.make_async_copy(
          out_hbm.at[pl.ds(0, 1), :], slots_v.at[pl.ds(0, 1), :],
          sem_out).wait()
      return 0

    drained = ((c2 - 1) // NSLOT) * NSLOT
    lax.fori_loop(0, c2 - drained, dbody, 0)

  def match_extract(q, src_ref, is_tail):
    def seg_body(s, _):
      seg0 = s * SEG
      segn = jnp.minimum(nh - seg0, SEG)
      c2 = match_q(q, seg0, segn, ge=is_tail)

      @pl.when(c2 > 0)
      def _():
        extract_hits(c2, src_ref, is_tail, 0)

      return 0

    lax.fori_loop(0, nseg, seg_body, 0)

  bufs = (bufa_v, bufb_v)
  sems = (sema, semb)

  def fire_half(hsb, half):
    start = pl.multiple_of(lo_col + hsb * HB, HB)
    pltpu.async_copy(table_hbm.at[:, pl.ds(start, HB)],
                     bufs[half], sems[half])

  fire_half(0, 0)
  fire_half(1, 1)

  def sweep_body(sb, _):
    for half in range(2):
      pltpu.make_async_copy(table_hbm.at[:, pl.ds(0, HB)],
                            bufs[half], sems[half]).wait()
      match_extract(2 * sb + half, bufs[half], False)

      @pl.when(sb < SB_PER_W - 1)
      def _():
        fire_half(2 * (sb + 1) + half, half)

    return 0

  lax.fori_loop(0, SB_PER_W, sweep_body, 0)

  @pl.when(is_last)
  def _():
    pltpu.sync_copy(tail_hbm, tailb_v)
    match_extract(TAIL_Q, tailb_v, True)


_sc_gather = pl.kernel(
    _sc_body,
    out_type=jax.ShapeDtypeStruct((BATCH, USER_EMBED_DIM), jnp.float32),
    mesh=plsc.VectorSubcoreMesh(core_axis_name="c", subcore_axis_name="s"),
    compiler_params=pltpu.CompilerParams(needs_layout_passes=False),
    scratch_types=[
        pltpu.VMEM((CAP,), jnp.int32),
        pltpu.VMEM((SCAP,), jnp.int32),
        pltpu.VMEM((N_CLUSTERS, HB), jnp.float32),
        pltpu.VMEM((N_CLUSTERS, HB), jnp.float32),
        pltpu.VMEM((TAIL_N, USER_EMBED_DIM), jnp.float32),
        pltpu.VMEM((NSLOT, USER_EMBED_DIM), jnp.float32),
        pltpu.SemaphoreType.DMA,
        pltpu.SemaphoreType.DMA,
        pltpu.SemaphoreType.DMA,
    ],
)


def _tc_body(rows_ref, ce_ref, out_ref):
  w = rows_ref[...]
  w = w - jnp.max(w, axis=1, keepdims=True)
  e = jnp.exp(w)
  p = e / jnp.sum(e, axis=1, keepdims=True)
  out_ref[...] = lax.dot_general(
      ce_ref[...], p, (((0,), (1,)), ((), ())),
      preferred_element_type=jnp.float32)


TC_BLOCK = 2048


def _tc_softmax_matmul(rows, ce):
  grid = BATCH // TC_BLOCK
  return pl.pallas_call(
      _tc_body,
      grid=(grid,),
      in_specs=[
          pl.BlockSpec((TC_BLOCK, N_CLUSTERS), lambda i: (i, 0)),
          pl.BlockSpec((N_CLUSTERS, USER_EMBED_DIM), lambda i: (0, 0)),
      ],
      out_specs=pl.BlockSpec((USER_EMBED_DIM, TC_BLOCK), lambda i: (0, i)),
      out_shape=jax.ShapeDtypeStruct((USER_EMBED_DIM, BATCH), jnp.float32),
  )(rows, ce)


@jax.jit
def kernel(user_identifiers, user_weight, cluster_embedding):
  idx = user_identifiers.astype(jnp.int32)
  table_t = jnp.swapaxes(user_weight, 0, 1)
  wt_tail = lax.slice(user_weight, (TAIL_START, 0), (N_USERS_P1, N_CLUSTERS))
  rows = _sc_gather(idx, table_t, wt_tail)
  out_t = _tc_softmax_matmul(rows, cluster_embedding)
  return jnp.swapaxes(out_t, 0, 1).reshape(BATCH, 1, USER_EMBED_DIM)
